# baseline (speedup 1.0000x reference)
import jax
import jax.numpy as jnp
from jax.experimental import pallas as pl
from jax.experimental.pallas import tpu as pltpu

_EPS = 1e-08
_ROWS = 8
_W = 256


def _halve_sum(v):
    w = v.shape[1]
    while w > 128:
        v = v[:, : w // 2] + v[:, w // 2:]
        w //= 2
    return v


def _stats_kernel(x_ref, o_ref):
    r, t = x_ref.shape
    nc = t // _W
    inv_t = 1.0 / t

    zeros = jnp.zeros((r, 128), jnp.float32)
    s1a = zeros
    s2a = zeros
    s3a = zeros
    s4a = zeros
    saa = zeros
    sqa = zeros
    mxa = jnp.full((r, 128), -jnp.inf, jnp.float32)
    mna = jnp.full((r, 128), jnp.inf, jnp.float32)

    for c in range(nc):
        lo = c * _W
        xc = x_ref[:, lo:lo + _W]
        x2 = xc * xc
        ax = jnp.abs(xc)
        s1a = s1a + _halve_sum(xc)
        s2a = s2a + _halve_sum(x2)
        s3a = s3a + _halve_sum(x2 * xc)
        s4a = s4a + _halve_sum(x2 * x2)
        saa = saa + _halve_sum(ax)
        sqa = sqa + _halve_sum(ax * jax.lax.rsqrt(ax + 1e-30))
        mxa = jnp.maximum(mxa, _halve_max(xc))
        mna = jnp.minimum(mna, _halve_min(xc))

    s1 = jnp.sum(s1a, axis=1, keepdims=True)
    mean = s1 * inv_t

    zca = jnp.zeros((r, 128), jnp.int32)
    mca = jnp.zeros((r, 128), jnp.int32)

    for c in range(nc):
        lo = c * _W
        xc = x_ref[:, lo:lo + _W]
        if c < nc - 1:
            xn = x_ref[:, lo + 1:lo + _W + 1]
        else:
            xn = jnp.concatenate(
                [x_ref[:, lo + 1:t], x_ref[:, t - 1:t]], axis=1)
        xi = pltpu.bitcast(xc, jnp.int32)
        xni = pltpu.bitcast(xn, jnp.int32)
        zi = jax.lax.shift_right_logical(xi ^ xni, 31)
        di = pltpu.bitcast(xc - mean, jnp.int32)
        dni = pltpu.bitcast(xn - mean, jnp.int32)
        mi = jax.lax.shift_right_logical(di ^ dni, 31)
        zca = zca + _halve_sum(zi)
        mca = mca + _halve_sum(mi)

    s2 = jnp.sum(s2a, axis=1, keepdims=True)
    s3 = jnp.sum(s3a, axis=1, keepdims=True)
    s4 = jnp.sum(s4a, axis=1, keepdims=True)
    sa = jnp.sum(saa, axis=1, keepdims=True)
    sq = jnp.sum(sqa, axis=1, keepdims=True)
    zc = jnp.sum(zca.astype(jnp.float32), axis=1, keepdims=True)
    mc = jnp.sum(mca.astype(jnp.float32), axis=1, keepdims=True)
    peak = jnp.max(mxa, axis=1, keepdims=True)
    peak_neg = jnp.min(mna, axis=1, keepdims=True)

    m2 = s2 * inv_t
    m3 = s3 * inv_t
    m4 = s4 * inv_t
    var = (s2 - s1 * mean) * (1.0 / (t - 1))
    std = jnp.sqrt(var)
    rms = jnp.sqrt(m2)
    energy = s2
    ptp = peak - peak_neg
    abs_peak = jnp.abs(peak)
    crest = abs_peak / (rms + _EPS)
    mean_abs = sa * inv_t
    shape_f = rms / (mean_abs + _EPS)
    impulse = abs_peak / (mean_abs + _EPS)
    sqrt_mean = sq * inv_t
    clearance = abs_peak / (sqrt_mean * sqrt_mean + _EPS)
    margin = abs_peak / (sqrt_mean + _EPS)

    mean2 = mean * mean
    m3c = m3 - 3.0 * mean * m2 + 2.0 * mean * mean2
    m4c = m4 - 4.0 * mean * m3 + 6.0 * mean2 * m2 - 3.0 * mean2 * mean2
    skew = m3c / (std * var + _EPS)
    kurt = m4c / (var * var + _EPS) - 3.0

    inv_tm1 = 1.0 / (t - 1)
    zcr = zc * inv_tm1
    mcr = mc * inv_tm1

    o_ref[...] = jnp.concatenate([
        mean, std, var, rms, peak, peak_neg, ptp,
        crest, shape_f, impulse, clearance,
        skew, kurt, zcr, mcr, margin, energy,
    ], axis=1)


def _halve_max(v):
    w = v.shape[1]
    while w > 128:
        v = jnp.maximum(v[:, : w // 2], v[:, w // 2:])
        w //= 2
    return v


def _halve_min(v):
    w = v.shape[1]
    while w > 128:
        v = jnp.minimum(v[:, : w // 2], v[:, w // 2:])
        w //= 2
    return v


def kernel(x):
    b, c, t = x.shape
    rows = b * c
    xr = x.reshape(rows, t)
    grid = (rows // _ROWS,)
    out = pl.pallas_call(
        _stats_kernel,
        grid=grid,
        in_specs=[pl.BlockSpec((_ROWS, t), lambda i: (i, 0))],
        out_specs=pl.BlockSpec((_ROWS, 17), lambda i: (i, 0)),
        out_shape=jax.ShapeDtypeStruct((rows, 17), jnp.float32),
        compiler_params=pltpu.CompilerParams(
            dimension_semantics=("arbitrary",),
        ),
    )(xr)
    return out.reshape(b, c, 17)


# speedup vs baseline: 1.0333x; 1.0333x over previous
import jax
import jax.numpy as jnp
from jax.experimental import pallas as pl
from jax.experimental.pallas import tpu as pltpu

_EPS = 1e-08
_ROWS = 8
_W = 256


def _halve_sum(v):
    w = v.shape[1]
    while w > 128:
        v = v[:, : w // 2] + v[:, w // 2:]
        w //= 2
    return v


def _halve_max(v):
    w = v.shape[1]
    while w > 128:
        v = jnp.maximum(v[:, : w // 2], v[:, w // 2:])
        w //= 2
    return v


def _halve_min(v):
    w = v.shape[1]
    while w > 128:
        v = jnp.minimum(v[:, : w // 2], v[:, w // 2:])
        w //= 2
    return v


def _stats_kernel(x_ref, o_ref):
    r, t = x_ref.shape
    nc = t // _W
    inv_t = 1.0 / t

    zeros = jnp.zeros((r, _W), jnp.float32)
    s1a = zeros
    s2a = zeros
    s3a = zeros
    s4a = zeros
    saa = zeros
    sqa = zeros
    mxa = jnp.full((r, _W), -jnp.inf, jnp.float32)
    mna = jnp.full((r, _W), jnp.inf, jnp.float32)

    for c in range(nc):
        lo = c * _W
        xc = x_ref[:, lo:lo + _W]
        x2 = xc * xc
        ax = jnp.abs(xc)
        s1a = s1a + xc
        s2a = s2a + x2
        s3a = s3a + x2 * xc
        s4a = s4a + x2 * x2
        saa = saa + ax
        sqa = sqa + ax * jax.lax.rsqrt(ax + 1e-30)
        mxa = jnp.maximum(mxa, xc)
        mna = jnp.minimum(mna, xc)

    s1 = jnp.sum(_halve_sum(s1a), axis=1, keepdims=True)
    mean = s1 * inv_t

    zca = jnp.zeros((r, _W), jnp.int32)
    mca = jnp.zeros((r, _W), jnp.int32)

    for c in range(nc):
        lo = c * _W
        xc = x_ref[:, lo:lo + _W]
        if c < nc - 1:
            xn = x_ref[:, lo + 1:lo + _W + 1]
        else:
            xn = jnp.concatenate(
                [x_ref[:, lo + 1:t], x_ref[:, t - 1:t]], axis=1)
        xi = pltpu.bitcast(xc, jnp.int32)
        xni = pltpu.bitcast(xn, jnp.int32)
        zca = zca + jax.lax.shift_right_logical(xi ^ xni, 31)
        di = pltpu.bitcast(xc - mean, jnp.int32)
        dni = pltpu.bitcast(xn - mean, jnp.int32)
        mca = mca + jax.lax.shift_right_logical(di ^ dni, 31)

    s2 = jnp.sum(_halve_sum(s2a), axis=1, keepdims=True)
    s3 = jnp.sum(_halve_sum(s3a), axis=1, keepdims=True)
    s4 = jnp.sum(_halve_sum(s4a), axis=1, keepdims=True)
    sa = jnp.sum(_halve_sum(saa), axis=1, keepdims=True)
    sq = jnp.sum(_halve_sum(sqa), axis=1, keepdims=True)
    zc = jnp.sum(_halve_sum(zca.astype(jnp.float32)), axis=1, keepdims=True)
    mc = jnp.sum(_halve_sum(mca.astype(jnp.float32)), axis=1, keepdims=True)
    peak = jnp.max(_halve_max(mxa), axis=1, keepdims=True)
    peak_neg = jnp.min(_halve_min(mna), axis=1, keepdims=True)

    m2 = s2 * inv_t
    m3 = s3 * inv_t
    m4 = s4 * inv_t
    var = (s2 - s1 * mean) * (1.0 / (t - 1))
    std = jnp.sqrt(var)
    rms = jnp.sqrt(m2)
    energy = s2
    ptp = peak - peak_neg
    abs_peak = jnp.abs(peak)
    crest = abs_peak / (rms + _EPS)
    mean_abs = sa * inv_t
    shape_f = rms / (mean_abs + _EPS)
    impulse = abs_peak / (mean_abs + _EPS)
    sqrt_mean = sq * inv_t
    clearance = abs_peak / (sqrt_mean * sqrt_mean + _EPS)
    margin = abs_peak / (sqrt_mean + _EPS)

    mean2 = mean * mean
    m3c = m3 - 3.0 * mean * m2 + 2.0 * mean * mean2
    m4c = m4 - 4.0 * mean * m3 + 6.0 * mean2 * m2 - 3.0 * mean2 * mean2
    skew = m3c / (std * var + _EPS)
    kurt = m4c / (var * var + _EPS) - 3.0

    inv_tm1 = 1.0 / (t - 1)
    zcr = zc * inv_tm1
    mcr = mc * inv_tm1

    o_ref[...] = jnp.concatenate([
        mean, std, var, rms, peak, peak_neg, ptp,
        crest, shape_f, impulse, clearance,
        skew, kurt, zcr, mcr, margin, energy,
    ], axis=1)


def kernel(x):
    b, c, t = x.shape
    rows = b * c
    xr = x.reshape(rows, t)
    grid = (rows // _ROWS,)
    out = pl.pallas_call(
        _stats_kernel,
        grid=grid,
        in_specs=[pl.BlockSpec((_ROWS, t), lambda i: (i, 0))],
        out_specs=pl.BlockSpec((_ROWS, 17), lambda i: (i, 0)),
        out_shape=jax.ShapeDtypeStruct((rows, 17), jnp.float32),
        compiler_params=pltpu.CompilerParams(
            dimension_semantics=("arbitrary",),
        ),
    )(xr)
    return out.reshape(b, c, 17)


# speedup vs baseline: 1.0383x; 1.0048x over previous
import jax
import jax.numpy as jnp
from jax.experimental import pallas as pl
from jax.experimental.pallas import tpu as pltpu

_EPS = 1e-08
_ROWS = 8
_W = 256


def _halve_sum(v):
    w = v.shape[1]
    while w > 128:
        v = v[:, : w // 2] + v[:, w // 2:]
        w //= 2
    return v


def _halve_max(v):
    w = v.shape[1]
    while w > 128:
        v = jnp.maximum(v[:, : w // 2], v[:, w // 2:])
        w //= 2
    return v


def _halve_min(v):
    w = v.shape[1]
    while w > 128:
        v = jnp.minimum(v[:, : w // 2], v[:, w // 2:])
        w //= 2
    return v


def _stats_kernel(x_ref, o_ref):
    r, t = x_ref.shape
    nc = t // _W
    inv_t = 1.0 / t

    zeros = jnp.zeros((r, _W), jnp.float32)
    s1a = zeros
    s2a = zeros
    s3a = zeros
    s4a = zeros
    saa = zeros
    sqa = zeros
    mxa = jnp.full((r, _W), -jnp.inf, jnp.float32)
    mna = jnp.full((r, _W), jnp.inf, jnp.float32)

    for c in range(nc):
        lo = c * _W
        xc = x_ref[:, lo:lo + _W]
        x2 = xc * xc
        ax = jnp.abs(xc)
        s1a = s1a + xc
        s2a = s2a + x2
        s3a = s3a + x2 * xc
        s4a = s4a + x2 * x2
        saa = saa + ax
        sqa = sqa + ax * jax.lax.rsqrt(ax + 1e-30)

    s1 = jnp.sum(_halve_sum(s1a), axis=1, keepdims=True)
    mean = s1 * inv_t

    zca = jnp.zeros((r, _W), jnp.int32)
    mca = jnp.zeros((r, _W), jnp.int32)

    for c in range(nc):
        lo = c * _W
        xc = x_ref[:, lo:lo + _W]
        if c < nc - 1:
            xn = x_ref[:, lo + 1:lo + _W + 1]
        else:
            xn = jnp.concatenate(
                [x_ref[:, lo + 1:t], x_ref[:, t - 1:t]], axis=1)
        xi = pltpu.bitcast(xc, jnp.int32)
        xni = pltpu.bitcast(xn, jnp.int32)
        zca = zca + jax.lax.shift_right_logical(xi ^ xni, 31)
        di = pltpu.bitcast(xc - mean, jnp.int32)
        dni = pltpu.bitcast(xn - mean, jnp.int32)
        mca = mca + jax.lax.shift_right_logical(di ^ dni, 31)
        mxa = jnp.maximum(mxa, xc)
        mna = jnp.minimum(mna, xc)

    s2 = jnp.sum(_halve_sum(s2a), axis=1, keepdims=True)
    s3 = jnp.sum(_halve_sum(s3a), axis=1, keepdims=True)
    s4 = jnp.sum(_halve_sum(s4a), axis=1, keepdims=True)
    sa = jnp.sum(_halve_sum(saa), axis=1, keepdims=True)
    sq = jnp.sum(_halve_sum(sqa), axis=1, keepdims=True)
    zc = jnp.sum(_halve_sum(zca.astype(jnp.float32)), axis=1, keepdims=True)
    mc = jnp.sum(_halve_sum(mca.astype(jnp.float32)), axis=1, keepdims=True)
    peak = jnp.max(_halve_max(mxa), axis=1, keepdims=True)
    peak_neg = jnp.min(_halve_min(mna), axis=1, keepdims=True)

    m2 = s2 * inv_t
    m3 = s3 * inv_t
    m4 = s4 * inv_t
    var = (s2 - s1 * mean) * (1.0 / (t - 1))
    std = jnp.sqrt(var)
    rms = jnp.sqrt(m2)
    energy = s2
    ptp = peak - peak_neg
    abs_peak = jnp.abs(peak)
    crest = abs_peak / (rms + _EPS)
    mean_abs = sa * inv_t
    shape_f = rms / (mean_abs + _EPS)
    impulse = abs_peak / (mean_abs + _EPS)
    sqrt_mean = sq * inv_t
    clearance = abs_peak / (sqrt_mean * sqrt_mean + _EPS)
    margin = abs_peak / (sqrt_mean + _EPS)

    mean2 = mean * mean
    m3c = m3 - 3.0 * mean * m2 + 2.0 * mean * mean2
    m4c = m4 - 4.0 * mean * m3 + 6.0 * mean2 * m2 - 3.0 * mean2 * mean2
    skew = m3c / (std * var + _EPS)
    kurt = m4c / (var * var + _EPS) - 3.0

    inv_tm1 = 1.0 / (t - 1)
    zcr = zc * inv_tm1
    mcr = mc * inv_tm1

    o_ref[...] = jnp.concatenate([
        mean, std, var, rms, peak, peak_neg, ptp,
        crest, shape_f, impulse, clearance,
        skew, kurt, zcr, mcr, margin, energy,
    ], axis=1)


def kernel(x):
    b, c, t = x.shape
    rows = b * c
    xr = x.reshape(rows, t)
    grid = (rows // _ROWS,)
    out = pl.pallas_call(
        _stats_kernel,
        grid=grid,
        in_specs=[pl.BlockSpec((_ROWS, t), lambda i: (i, 0))],
        out_specs=pl.BlockSpec((_ROWS, 17), lambda i: (i, 0)),
        out_shape=jax.ShapeDtypeStruct((rows, 17), jnp.float32),
        compiler_params=pltpu.CompilerParams(
            dimension_semantics=("arbitrary",),
        ),
    )(xr)
    return out.reshape(b, c, 17)


# speedup vs baseline: 1.1139x; 1.0728x over previous
import jax
import jax.numpy as jnp
from jax.experimental import pallas as pl
from jax.experimental.pallas import tpu as pltpu

_EPS = 1e-08
_ROWS = 8
_W = 256


def _halve_sum(v):
    w = v.shape[1]
    while w > 128:
        v = v[:, : w // 2] + v[:, w // 2:]
        w //= 2
    return v


def _halve_max(v):
    w = v.shape[1]
    while w > 128:
        v = jnp.maximum(v[:, : w // 2], v[:, w // 2:])
        w //= 2
    return v


def _halve_min(v):
    w = v.shape[1]
    while w > 128:
        v = jnp.minimum(v[:, : w // 2], v[:, w // 2:])
        w //= 2
    return v


def _stats_kernel(x_ref, o_ref):
    r, t = x_ref.shape
    nc = t // _W
    inv_t = 1.0 / t

    zeros = jnp.zeros((r, _W), jnp.float32)
    s1a = zeros
    s2a = zeros
    s3a = zeros
    s4a = zeros
    saa = zeros
    sqa = zeros
    mxa = jnp.full((r, _W), -jnp.inf, jnp.float32)
    mna = jnp.full((r, _W), jnp.inf, jnp.float32)
    zca = jnp.zeros((r, _W), jnp.int32)

    for c in range(nc):
        lo = c * _W
        xc = x_ref[:, lo:lo + _W]
        x2 = xc * xc
        ax = jnp.abs(xc)
        s1a = s1a + xc
        s2a = s2a + x2
        s3a = s3a + x2 * xc
        s4a = s4a + x2 * x2
        saa = saa + ax
        sqa = sqa + ax * jax.lax.rsqrt(ax + 1e-30)
        mxa = jnp.maximum(mxa, xc)
        mna = jnp.minimum(mna, xc)
        if c < nc - 1:
            xn = x_ref[:, lo + 1:lo + _W + 1]
        else:
            xn = jnp.concatenate(
                [x_ref[:, lo + 1:t], x_ref[:, t - 1:t]], axis=1)
        xi = pltpu.bitcast(xc, jnp.int32)
        xni = pltpu.bitcast(xn, jnp.int32)
        zca = zca + jax.lax.shift_right_logical(xi ^ xni, 31)

    s1 = jnp.sum(_halve_sum(s1a), axis=1, keepdims=True)
    mean = s1 * inv_t

    mca = jnp.zeros((r, _W), jnp.int32)

    for c in range(nc):
        lo = c * _W
        xc = x_ref[:, lo:lo + _W]
        if c < nc - 1:
            xn = x_ref[:, lo + 1:lo + _W + 1]
        else:
            xn = jnp.concatenate(
                [x_ref[:, lo + 1:t], x_ref[:, t - 1:t]], axis=1)
        di = pltpu.bitcast(xc - mean, jnp.int32)
        dni = pltpu.bitcast(xn - mean, jnp.int32)
        mca = mca + jax.lax.shift_right_logical(di ^ dni, 31)

    s2 = jnp.sum(_halve_sum(s2a), axis=1, keepdims=True)
    s3 = jnp.sum(_halve_sum(s3a), axis=1, keepdims=True)
    s4 = jnp.sum(_halve_sum(s4a), axis=1, keepdims=True)
    sa = jnp.sum(_halve_sum(saa), axis=1, keepdims=True)
    sq = jnp.sum(_halve_sum(sqa), axis=1, keepdims=True)
    zc = jnp.sum(_halve_sum(zca.astype(jnp.float32)), axis=1, keepdims=True)
    mc = jnp.sum(_halve_sum(mca.astype(jnp.float32)), axis=1, keepdims=True)
    peak = jnp.max(_halve_max(mxa), axis=1, keepdims=True)
    peak_neg = jnp.min(_halve_min(mna), axis=1, keepdims=True)

    m2 = s2 * inv_t
    m3 = s3 * inv_t
    m4 = s4 * inv_t
    var = (s2 - s1 * mean) * (1.0 / (t - 1))
    std = jnp.sqrt(var)
    rms = jnp.sqrt(m2)
    energy = s2
    ptp = peak - peak_neg
    abs_peak = jnp.abs(peak)
    crest = abs_peak / (rms + _EPS)
    mean_abs = sa * inv_t
    shape_f = rms / (mean_abs + _EPS)
    impulse = abs_peak / (mean_abs + _EPS)
    sqrt_mean = sq * inv_t
    clearance = abs_peak / (sqrt_mean * sqrt_mean + _EPS)
    margin = abs_peak / (sqrt_mean + _EPS)

    mean2 = mean * mean
    m3c = m3 - 3.0 * mean * m2 + 2.0 * mean * mean2
    m4c = m4 - 4.0 * mean * m3 + 6.0 * mean2 * m2 - 3.0 * mean2 * mean2
    skew = m3c / (std * var + _EPS)
    kurt = m4c / (var * var + _EPS) - 3.0

    inv_tm1 = 1.0 / (t - 1)
    zcr = zc * inv_tm1
    mcr = mc * inv_tm1

    o_ref[...] = jnp.concatenate([
        mean, std, var, rms, peak, peak_neg, ptp,
        crest, shape_f, impulse, clearance,
        skew, kurt, zcr, mcr, margin, energy,
    ], axis=1)


def kernel(x):
    b, c, t = x.shape
    rows = b * c
    xr = x.reshape(rows, t)
    grid = (rows // _ROWS,)
    out = pl.pallas_call(
        _stats_kernel,
        grid=grid,
        in_specs=[pl.BlockSpec((_ROWS, t), lambda i: (i, 0))],
        out_specs=pl.BlockSpec((_ROWS, 17), lambda i: (i, 0)),
        out_shape=jax.ShapeDtypeStruct((rows, 17), jnp.float32),
        compiler_params=pltpu.CompilerParams(
            dimension_semantics=("arbitrary",),
        ),
    )(xr)
    return out.reshape(b, c, 17)


# speedup vs baseline: 1.1293x; 1.0139x over previous
import jax
import jax.numpy as jnp
from jax.experimental import pallas as pl
from jax.experimental.pallas import tpu as pltpu

_EPS = 1e-08
_ROWS = 8
_W = 128


def _halve_sum(v):
    w = v.shape[1]
    while w > 128:
        v = v[:, : w // 2] + v[:, w // 2:]
        w //= 2
    return v


def _halve_max(v):
    w = v.shape[1]
    while w > 128:
        v = jnp.maximum(v[:, : w // 2], v[:, w // 2:])
        w //= 2
    return v


def _halve_min(v):
    w = v.shape[1]
    while w > 128:
        v = jnp.minimum(v[:, : w // 2], v[:, w // 2:])
        w //= 2
    return v


def _stats_kernel(x_ref, o_ref):
    r, t = x_ref.shape
    nc = t // _W
    inv_t = 1.0 / t

    zeros = jnp.zeros((r, _W), jnp.float32)
    s1a = zeros
    s2a = zeros
    s3a = zeros
    s4a = zeros
    saa = zeros
    sqa = zeros
    mxa = jnp.full((r, _W), -jnp.inf, jnp.float32)
    mna = jnp.full((r, _W), jnp.inf, jnp.float32)
    zca = jnp.zeros((r, _W), jnp.int32)

    for c in range(nc):
        lo = c * _W
        xc = x_ref[:, lo:lo + _W]
        x2 = xc * xc
        ax = jnp.abs(xc)
        s1a = s1a + xc
        s2a = s2a + x2
        s3a = s3a + x2 * xc
        s4a = s4a + x2 * x2
        saa = saa + ax
        sqa = sqa + ax * jax.lax.rsqrt(ax + 1e-30)
        mxa = jnp.maximum(mxa, xc)
        mna = jnp.minimum(mna, xc)
        if c < nc - 1:
            xn = x_ref[:, lo + 1:lo + _W + 1]
        else:
            xn = jnp.concatenate(
                [x_ref[:, lo + 1:t], x_ref[:, t - 1:t]], axis=1)
        xi = pltpu.bitcast(xc, jnp.int32)
        xni = pltpu.bitcast(xn, jnp.int32)
        zca = zca + jax.lax.shift_right_logical(xi ^ xni, 31)

    s1 = jnp.sum(_halve_sum(s1a), axis=1, keepdims=True)
    mean = s1 * inv_t

    mca = jnp.zeros((r, _W), jnp.int32)

    for c in range(nc):
        lo = c * _W
        xc = x_ref[:, lo:lo + _W]
        if c < nc - 1:
            xn = x_ref[:, lo + 1:lo + _W + 1]
        else:
            xn = jnp.concatenate(
                [x_ref[:, lo + 1:t], x_ref[:, t - 1:t]], axis=1)
        di = pltpu.bitcast(xc - mean, jnp.int32)
        dni = pltpu.bitcast(xn - mean, jnp.int32)
        mca = mca + jax.lax.shift_right_logical(di ^ dni, 31)

    s2 = jnp.sum(_halve_sum(s2a), axis=1, keepdims=True)
    s3 = jnp.sum(_halve_sum(s3a), axis=1, keepdims=True)
    s4 = jnp.sum(_halve_sum(s4a), axis=1, keepdims=True)
    sa = jnp.sum(_halve_sum(saa), axis=1, keepdims=True)
    sq = jnp.sum(_halve_sum(sqa), axis=1, keepdims=True)
    zc = jnp.sum(_halve_sum(zca.astype(jnp.float32)), axis=1, keepdims=True)
    mc = jnp.sum(_halve_sum(mca.astype(jnp.float32)), axis=1, keepdims=True)
    peak = jnp.max(_halve_max(mxa), axis=1, keepdims=True)
    peak_neg = jnp.min(_halve_min(mna), axis=1, keepdims=True)

    m2 = s2 * inv_t
    m3 = s3 * inv_t
    m4 = s4 * inv_t
    var = (s2 - s1 * mean) * (1.0 / (t - 1))
    std = jnp.sqrt(var)
    rms = jnp.sqrt(m2)
    energy = s2
    ptp = peak - peak_neg
    abs_peak = jnp.abs(peak)
    crest = abs_peak / (rms + _EPS)
    mean_abs = sa * inv_t
    shape_f = rms / (mean_abs + _EPS)
    impulse = abs_peak / (mean_abs + _EPS)
    sqrt_mean = sq * inv_t
    clearance = abs_peak / (sqrt_mean * sqrt_mean + _EPS)
    margin = abs_peak / (sqrt_mean + _EPS)

    mean2 = mean * mean
    m3c = m3 - 3.0 * mean * m2 + 2.0 * mean * mean2
    m4c = m4 - 4.0 * mean * m3 + 6.0 * mean2 * m2 - 3.0 * mean2 * mean2
    skew = m3c / (std * var + _EPS)
    kurt = m4c / (var * var + _EPS) - 3.0

    inv_tm1 = 1.0 / (t - 1)
    zcr = zc * inv_tm1
    mcr = mc * inv_tm1

    o_ref[...] = jnp.concatenate([
        mean, std, var, rms, peak, peak_neg, ptp,
        crest, shape_f, impulse, clearance,
        skew, kurt, zcr, mcr, margin, energy,
    ], axis=1)


def kernel(x):
    b, c, t = x.shape
    rows = b * c
    xr = x.reshape(rows, t)
    grid = (rows // _ROWS,)
    out = pl.pallas_call(
        _stats_kernel,
        grid=grid,
        in_specs=[pl.BlockSpec((_ROWS, t), lambda i: (i, 0))],
        out_specs=pl.BlockSpec((_ROWS, 17), lambda i: (i, 0)),
        out_shape=jax.ShapeDtypeStruct((rows, 17), jnp.float32),
        compiler_params=pltpu.CompilerParams(
            dimension_semantics=("arbitrary",),
        ),
    )(xr)
    return out.reshape(b, c, 17)


# speedup vs baseline: 1.1364x; 1.0063x over previous
import jax
import jax.numpy as jnp
from jax.experimental import pallas as pl
from jax.experimental.pallas import tpu as pltpu

_EPS = 1e-08
_ROWS = 16
_W = 128


def _halve_sum(v):
    w = v.shape[1]
    while w > 128:
        v = v[:, : w // 2] + v[:, w // 2:]
        w //= 2
    return v


def _halve_max(v):
    w = v.shape[1]
    while w > 128:
        v = jnp.maximum(v[:, : w // 2], v[:, w // 2:])
        w //= 2
    return v


def _halve_min(v):
    w = v.shape[1]
    while w > 128:
        v = jnp.minimum(v[:, : w // 2], v[:, w // 2:])
        w //= 2
    return v


def _stats_kernel(x_ref, o_ref):
    r, t = x_ref.shape
    nc = t // _W
    inv_t = 1.0 / t

    zeros = jnp.zeros((r, _W), jnp.float32)
    s1a = zeros
    s2a = zeros
    s3a = zeros
    s4a = zeros
    saa = zeros
    sqa = zeros
    mxa = jnp.full((r, _W), -jnp.inf, jnp.float32)
    mna = jnp.full((r, _W), jnp.inf, jnp.float32)
    zca = jnp.zeros((r, _W), jnp.int32)

    for c in range(nc):
        lo = c * _W
        xc = x_ref[:, lo:lo + _W]
        x2 = xc * xc
        ax = jnp.abs(xc)
        s1a = s1a + xc
        s2a = s2a + x2
        s3a = s3a + x2 * xc
        s4a = s4a + x2 * x2
        saa = saa + ax
        sqa = sqa + ax * jax.lax.rsqrt(ax + 1e-30)
        mxa = jnp.maximum(mxa, xc)
        mna = jnp.minimum(mna, xc)
        if c < nc - 1:
            xn = x_ref[:, lo + 1:lo + _W + 1]
        else:
            xn = jnp.concatenate(
                [x_ref[:, lo + 1:t], x_ref[:, t - 1:t]], axis=1)
        xi = pltpu.bitcast(xc, jnp.int32)
        xni = pltpu.bitcast(xn, jnp.int32)
        zca = zca + jax.lax.shift_right_logical(xi ^ xni, 31)

    s1 = jnp.sum(_halve_sum(s1a), axis=1, keepdims=True)
    mean = s1 * inv_t

    mca = jnp.zeros((r, _W), jnp.int32)

    for c in range(nc):
        lo = c * _W
        xc = x_ref[:, lo:lo + _W]
        if c < nc - 1:
            xn = x_ref[:, lo + 1:lo + _W + 1]
        else:
            xn = jnp.concatenate(
                [x_ref[:, lo + 1:t], x_ref[:, t - 1:t]], axis=1)
        di = pltpu.bitcast(xc - mean, jnp.int32)
        dni = pltpu.bitcast(xn - mean, jnp.int32)
        mca = mca + jax.lax.shift_right_logical(di ^ dni, 31)

    s2 = jnp.sum(_halve_sum(s2a), axis=1, keepdims=True)
    s3 = jnp.sum(_halve_sum(s3a), axis=1, keepdims=True)
    s4 = jnp.sum(_halve_sum(s4a), axis=1, keepdims=True)
    sa = jnp.sum(_halve_sum(saa), axis=1, keepdims=True)
    sq = jnp.sum(_halve_sum(sqa), axis=1, keepdims=True)
    zc = jnp.sum(_halve_sum(zca.astype(jnp.float32)), axis=1, keepdims=True)
    mc = jnp.sum(_halve_sum(mca.astype(jnp.float32)), axis=1, keepdims=True)
    peak = jnp.max(_halve_max(mxa), axis=1, keepdims=True)
    peak_neg = jnp.min(_halve_min(mna), axis=1, keepdims=True)

    m2 = s2 * inv_t
    m3 = s3 * inv_t
    m4 = s4 * inv_t
    var = (s2 - s1 * mean) * (1.0 / (t - 1))
    std = jnp.sqrt(var)
    rms = jnp.sqrt(m2)
    energy = s2
    ptp = peak - peak_neg
    abs_peak = jnp.abs(peak)
    crest = abs_peak / (rms + _EPS)
    mean_abs = sa * inv_t
    shape_f = rms / (mean_abs + _EPS)
    impulse = abs_peak / (mean_abs + _EPS)
    sqrt_mean = sq * inv_t
    clearance = abs_peak / (sqrt_mean * sqrt_mean + _EPS)
    margin = abs_peak / (sqrt_mean + _EPS)

    mean2 = mean * mean
    m3c = m3 - 3.0 * mean * m2 + 2.0 * mean * mean2
    m4c = m4 - 4.0 * mean * m3 + 6.0 * mean2 * m2 - 3.0 * mean2 * mean2
    skew = m3c / (std * var + _EPS)
    kurt = m4c / (var * var + _EPS) - 3.0

    inv_tm1 = 1.0 / (t - 1)
    zcr = zc * inv_tm1
    mcr = mc * inv_tm1

    o_ref[...] = jnp.concatenate([
        mean, std, var, rms, peak, peak_neg, ptp,
        crest, shape_f, impulse, clearance,
        skew, kurt, zcr, mcr, margin, energy,
    ], axis=1)


def kernel(x):
    b, c, t = x.shape
    rows = b * c
    xr = x.reshape(rows, t)
    grid = (rows // _ROWS,)
    out = pl.pallas_call(
        _stats_kernel,
        grid=grid,
        in_specs=[pl.BlockSpec((_ROWS, t), lambda i: (i, 0))],
        out_specs=pl.BlockSpec((_ROWS, 17), lambda i: (i, 0)),
        out_shape=jax.ShapeDtypeStruct((rows, 17), jnp.float32),
        compiler_params=pltpu.CompilerParams(
            dimension_semantics=("arbitrary",),
        ),
    )(xr)
    return out.reshape(b, c, 17)


# speedup vs baseline: 1.1702x; 1.0297x over previous
import jax
import jax.numpy as jnp
from jax.experimental import pallas as pl
from jax.experimental.pallas import tpu as pltpu

_EPS = 1e-08
_ROWS = 32
_W = 128


def _halve_sum(v):
    w = v.shape[1]
    while w > 128:
        v = v[:, : w // 2] + v[:, w // 2:]
        w //= 2
    return v


def _halve_max(v):
    w = v.shape[1]
    while w > 128:
        v = jnp.maximum(v[:, : w // 2], v[:, w // 2:])
        w //= 2
    return v


def _halve_min(v):
    w = v.shape[1]
    while w > 128:
        v = jnp.minimum(v[:, : w // 2], v[:, w // 2:])
        w //= 2
    return v


def _stats_kernel(x_ref, o_ref):
    r, t = x_ref.shape
    nc = t // _W
    inv_t = 1.0 / t

    zeros = jnp.zeros((r, _W), jnp.float32)
    s1a = zeros
    s2a = zeros
    s3a = zeros
    s4a = zeros
    saa = zeros
    sqa = zeros
    mxa = jnp.full((r, _W), -jnp.inf, jnp.float32)
    mna = jnp.full((r, _W), jnp.inf, jnp.float32)
    zca = jnp.zeros((r, _W), jnp.int32)

    for c in range(nc):
        lo = c * _W
        xc = x_ref[:, lo:lo + _W]
        x2 = xc * xc
        ax = jnp.abs(xc)
        s1a = s1a + xc
        s2a = s2a + x2
        s3a = s3a + x2 * xc
        s4a = s4a + x2 * x2
        saa = saa + ax
        sqa = sqa + ax * jax.lax.rsqrt(ax + 1e-30)
        mxa = jnp.maximum(mxa, xc)
        mna = jnp.minimum(mna, xc)
        if c < nc - 1:
            xn = x_ref[:, lo + 1:lo + _W + 1]
        else:
            xn = jnp.concatenate(
                [x_ref[:, lo + 1:t], x_ref[:, t - 1:t]], axis=1)
        xi = pltpu.bitcast(xc, jnp.int32)
        xni = pltpu.bitcast(xn, jnp.int32)
        zca = zca + jax.lax.shift_right_logical(xi ^ xni, 31)

    s1 = jnp.sum(_halve_sum(s1a), axis=1, keepdims=True)
    mean = s1 * inv_t

    mca = jnp.zeros((r, _W), jnp.int32)

    for c in range(nc):
        lo = c * _W
        xc = x_ref[:, lo:lo + _W]
        if c < nc - 1:
            xn = x_ref[:, lo + 1:lo + _W + 1]
        else:
            xn = jnp.concatenate(
                [x_ref[:, lo + 1:t], x_ref[:, t - 1:t]], axis=1)
        di = pltpu.bitcast(xc - mean, jnp.int32)
        dni = pltpu.bitcast(xn - mean, jnp.int32)
        mca = mca + jax.lax.shift_right_logical(di ^ dni, 31)

    s2 = jnp.sum(_halve_sum(s2a), axis=1, keepdims=True)
    s3 = jnp.sum(_halve_sum(s3a), axis=1, keepdims=True)
    s4 = jnp.sum(_halve_sum(s4a), axis=1, keepdims=True)
    sa = jnp.sum(_halve_sum(saa), axis=1, keepdims=True)
    sq = jnp.sum(_halve_sum(sqa), axis=1, keepdims=True)
    zc = jnp.sum(_halve_sum(zca.astype(jnp.float32)), axis=1, keepdims=True)
    mc = jnp.sum(_halve_sum(mca.astype(jnp.float32)), axis=1, keepdims=True)
    peak = jnp.max(_halve_max(mxa), axis=1, keepdims=True)
    peak_neg = jnp.min(_halve_min(mna), axis=1, keepdims=True)

    m2 = s2 * inv_t
    m3 = s3 * inv_t
    m4 = s4 * inv_t
    var = (s2 - s1 * mean) * (1.0 / (t - 1))
    std = jnp.sqrt(var)
    rms = jnp.sqrt(m2)
    energy = s2
    ptp = peak - peak_neg
    abs_peak = jnp.abs(peak)
    crest = abs_peak / (rms + _EPS)
    mean_abs = sa * inv_t
    shape_f = rms / (mean_abs + _EPS)
    impulse = abs_peak / (mean_abs + _EPS)
    sqrt_mean = sq * inv_t
    clearance = abs_peak / (sqrt_mean * sqrt_mean + _EPS)
    margin = abs_peak / (sqrt_mean + _EPS)

    mean2 = mean * mean
    m3c = m3 - 3.0 * mean * m2 + 2.0 * mean * mean2
    m4c = m4 - 4.0 * mean * m3 + 6.0 * mean2 * m2 - 3.0 * mean2 * mean2
    skew = m3c / (std * var + _EPS)
    kurt = m4c / (var * var + _EPS) - 3.0

    inv_tm1 = 1.0 / (t - 1)
    zcr = zc * inv_tm1
    mcr = mc * inv_tm1

    o_ref[...] = jnp.concatenate([
        mean, std, var, rms, peak, peak_neg, ptp,
        crest, shape_f, impulse, clearance,
        skew, kurt, zcr, mcr, margin, energy,
    ], axis=1)


def kernel(x):
    b, c, t = x.shape
    rows = b * c
    xr = x.reshape(rows, t)
    grid = (rows // _ROWS,)
    out = pl.pallas_call(
        _stats_kernel,
        grid=grid,
        in_specs=[pl.BlockSpec((_ROWS, t), lambda i: (i, 0))],
        out_specs=pl.BlockSpec((_ROWS, 17), lambda i: (i, 0)),
        out_shape=jax.ShapeDtypeStruct((rows, 17), jnp.float32),
        compiler_params=pltpu.CompilerParams(
            dimension_semantics=("arbitrary",),
        ),
    )(xr)
    return out.reshape(b, c, 17)
